# Initial kernel scaffold; baseline (speedup 1.0000x reference)
import jax
import jax.numpy as jnp
from jax.experimental import pallas as pl


def kernel(x_i, fg_sdf, bg_sdf):
    raise NotImplementedError("write your pallas kernel here")


# speedup vs baseline: 2.8069x; 2.8069x over previous
import jax
import jax.numpy as jnp
from jax import lax
from jax.experimental import pallas as pl
from jax.experimental.pallas import tpu as pltpu
from jax.experimental.pallas import tpu_sc as plsc

NC, NS, L = 2, 16, 16
NW = NC * NS
P = 4 * 65536
PER_W = P // NW
C = 2048
NCHUNK = PER_W // C
NG = C // L
R = 256
RM1 = float(R - 1)


def _body(pts_hbm, fg_hbm, bg_hbm, out_hbm,
          ptv0, idx0, fv0, bv0, outv0,
          ptv1, idx1, fv1, bv1, outv1,
          semf0, semf1, semb0, semb1):
    wid = lax.axis_index("s") * NC + lax.axis_index("c")
    wbase = wid * PER_W
    ptv = (ptv0, ptv1)
    idx = (idx0, idx1); fv = (fv0, fv1); bv = (bv0, bv1)
    outv = (outv0, outv1)
    semf = (semf0, semf1)
    semb = (semb0, semb1)

    def coords(b, i):
        o = i * (3 * L)
        va = ptv[b][pl.ds(o, L)]
        vb = ptv[b][pl.ds(o + L, L)]
        vc = ptv[b][pl.ds(o + 2 * L, L)]
        lane = lax.iota(jnp.int32, L)
        j3 = lane * 3

        dn = lax.GatherDimensionNumbers(offset_dims=(), collapsed_slice_dims=(0,),
                                        start_index_map=(0,))

        def vgather(v, i16):
            return lax.gather(v, i16[:, None], dn, (1,),
                              mode=lax.GatherScatterMode.PROMISE_IN_BOUNDS)

        def deint(off):
            p = (j3 + off) & (L - 1)
            ga = vgather(va, p)
            gb = vgather(vb, p)
            gc = vgather(vc, p)
            g = j3 + off
            return jnp.where(g < L, ga, jnp.where(g < 2 * L, gb, gc))

        px = deint(0)
        py = deint(1)
        pz = deint(2)
        inf = ((px > -1.0) & (px < 1.0) & (py > -1.0) & (py < 1.0)
               & (pz > -1.0) & (pz < 1.0))
        inbig = ((px > -4.0) & (px < 4.0) & (py > -4.0) & (py < 4.0)
                 & (pz > -4.0) & (pz < 4.0))
        scale = jnp.where(inf, 1.0, 0.25)
        xw = jnp.clip((pz * scale + 1.0) * 0.5 * RM1, 0.0, RM1)
        yh = jnp.clip((py * scale + 1.0) * 0.5 * RM1, 0.0, RM1)
        zd = jnp.clip((px * scale + 1.0) * 0.5 * RM1, 0.0, RM1)
        x0 = xw.astype(jnp.int32)
        y0 = yh.astype(jnp.int32)
        z0 = zd.astype(jnp.int32)
        wx = xw - x0.astype(jnp.float32)
        wy = yh - y0.astype(jnp.float32)
        wz = zd - z0.astype(jnp.float32)
        return inf, inbig, x0, y0, z0, wx, wy, wz

    def pass1(b):
        def body(i, _):
            _, _, x0, y0, z0, wx, wy, wz = coords(b, i)
            x1 = jnp.minimum(x0 + 1, R - 1)
            y1 = jnp.minimum(y0 + 1, R - 1)
            z1 = jnp.minimum(z0 + 1, R - 1)
            row0 = z0 * (R * R) + y0 * R
            row1 = z0 * (R * R) + y1 * R
            row2 = z1 * (R * R) + y0 * R
            row3 = z1 * (R * R) + y1 * R
            s = i * (8 * L)
            idx[b][pl.ds(s + 0 * L, L)] = row0 + x0
            idx[b][pl.ds(s + 1 * L, L)] = row0 + x1
            idx[b][pl.ds(s + 2 * L, L)] = row1 + x0
            idx[b][pl.ds(s + 3 * L, L)] = row1 + x1
            idx[b][pl.ds(s + 4 * L, L)] = row2 + x0
            idx[b][pl.ds(s + 5 * L, L)] = row2 + x1
            idx[b][pl.ds(s + 6 * L, L)] = row3 + x0
            idx[b][pl.ds(s + 7 * L, L)] = row3 + x1
            return 0

        lax.fori_loop(0, NG, body, 0)

    def pass2(b):
        def body(i, _):
            inf, inbig, _, _, _, wx, wy, wz = coords(b, i)
            s = i * (8 * L)

            def corner(k):
                return jnp.where(inf, fv[b][pl.ds(s + k * L, L)],
                                 bv[b][pl.ds(s + k * L, L)])

            c000 = corner(0); c001 = corner(1)
            c010 = corner(2); c011 = corner(3)
            c100 = corner(4); c101 = corner(5)
            c110 = corner(6); c111 = corner(7)
            c00 = c000 * (1.0 - wx) + c001 * wx
            c01 = c010 * (1.0 - wx) + c011 * wx
            c10 = c100 * (1.0 - wx) + c101 * wx
            c11 = c110 * (1.0 - wx) + c111 * wx
            c0 = c00 * (1.0 - wy) + c01 * wy
            c1 = c10 * (1.0 - wy) + c11 * wy
            val = c0 * (1.0 - wz) + c1 * wz
            outv[b][pl.ds(i * L, L)] = jnp.where(inbig, val, 1.0)
            return 0

        lax.fori_loop(0, NG, body, 0)

    prev = None
    for c in range(NCHUNK):
        b = c % 2
        base = wbase + c * C
        pltpu.sync_copy(pts_hbm.at[pl.ds(base * 3, C * 3)], ptv[b])
        pass1(b)
        cf = pltpu.async_copy(fg_hbm.at[idx[b]], fv[b], semf[b])
        cb = pltpu.async_copy(bg_hbm.at[idx[b]], bv[b], semb[b])
        if prev is not None:
            pcf, pcb, pbase, pb = prev
            pcf.wait()
            pcb.wait()
            pass2(pb)
            pltpu.sync_copy(outv[pb], out_hbm.at[pl.ds(pbase, C)])
        prev = (cf, cb, base, b)

    pcf, pcb, pbase, pb = prev
    pcf.wait()
    pcb.wait()
    pass2(pb)
    pltpu.sync_copy(outv[pb], out_hbm.at[pl.ds(pbase, C)])


@jax.jit
def kernel(x_i, fg_sdf, bg_sdf):
    B, N = x_i.shape[0], x_i.shape[1]
    pts = x_i.reshape(-1)
    fg = fg_sdf.reshape(-1)
    bg = bg_sdf.reshape(-1)
    mesh = plsc.VectorSubcoreMesh(core_axis_name="c", subcore_axis_name="s",
                                  num_cores=NC, num_subcores=NS)
    run = pl.kernel(
        _body,
        out_type=jax.ShapeDtypeStruct((P,), jnp.float32),
        mesh=mesh,
        scratch_types=(
            [pltpu.VMEM((3 * C,), jnp.float32),
             pltpu.VMEM((8 * C,), jnp.int32),
             pltpu.VMEM((8 * C,), jnp.float32),
             pltpu.VMEM((8 * C,), jnp.float32),
             pltpu.VMEM((C,), jnp.float32)]
        ) * 2 + [pltpu.SemaphoreType.DMA] * 4,
    )
    out = run(pts, fg, bg)
    return out.reshape(B, N)
